# baseline (speedup 1.0000x reference)
import functools

import jax
import jax.numpy as jnp
from jax import lax
from jax.experimental import pallas as pl
from jax.experimental.pallas import tpu as pltpu
from jax.experimental.pallas import tpu_sc as plsc

A = 2
NODE_NUM = 8
BS = 1024
EMB = 256
HID = 256
NREL = 5
NBASES = 4
E = 65536
N = NODE_NUM * BS

NC = 2
NS = 16
HALF = N // NC
EPT = E // NS
CAP = EPT
CHUNK = 128
NCHUNKS = CAP // CHUNK
RPT = HALF // NS
PPT = BS // NS
HKEYS = HALF * 8
HHALF = HKEYS // NS

_ZEROS16F = None


def _mesh():
    return plsc.VectorSubcoreMesh(core_axis_name="c", subcore_axis_name="s")


def _zero_rows(ref, nrows):
    z = jnp.zeros((16,), jnp.float32)

    def body(i, _):
        for k in range(16):
            ref[i, pl.ds(k * 16, 16)] = z
        return 0

    lax.fori_loop(0, nrows, body, 0)


def _preprocess_body(src_hbm, dst_hbm, et_hbm,
                     srcl_hbm, rowl_hbm, counts_hbm, rcp_hbm,
                     src_v, dst_v, et_v, hist_v, srcf, rowf, rowl2,
                     counts_v, acc_v, tmp_v, hshared):
    cid = lax.axis_index("c")
    sid = lax.axis_index("s")
    base = sid * EPT

    pltpu.sync_copy(src_hbm.at[pl.ds(base, EPT)], src_v)
    pltpu.sync_copy(dst_hbm.at[pl.ds(base, EPT)], dst_v)
    pltpu.sync_copy(et_hbm.at[pl.ds(base, EPT)], et_v)

    zf = jnp.zeros((16,), jnp.float32)
    zi = jnp.zeros((16,), jnp.int32)

    def zh(i, _):
        hist_v[pl.ds(i * 16, 16)] = zf
        return 0
    lax.fori_loop(0, HKEYS // 16, zh, 0)

    def zs(i, _):
        srcf[pl.ds(i * 16, 16)] = zi
        return 0
    lax.fori_loop(0, CAP // 16, zs, 0)

    ones = jnp.ones((16,), jnp.float32)

    def hb(g, _):
        d16 = dst_v[pl.ds(g * 16, 16)]
        e16 = et_v[pl.ds(g * 16, 16)]
        m = (d16 >> 12) == cid
        ploc = d16 & (HALF - 1)
        key = ploc * 8 + e16
        key = jnp.where(m, key, 0)
        plsc.addupdate_scatter(hist_v, [key], ones, mask=m)
        return 0
    lax.fori_loop(0, EPT // 16, hb, 0)

    padrow = jnp.full((16,), HALF, jnp.int32)
    for r in range(NREL):
        def rf(i, _):
            rowf[pl.ds(i * 16, 16)] = padrow
            return 0
        lax.fori_loop(0, CAP // 16, rf, 0)

        def cb(g, wp):
            d16 = dst_v[pl.ds(g * 16, 16)]
            e16 = et_v[pl.ds(g * 16, 16)]
            s16 = src_v[pl.ds(g * 16, 16)]
            m = ((d16 >> 12) == cid) & (e16 == r)
            cs = plsc.cumsum(m.astype(jnp.int32))
            pos = wp + cs - 1
            pos = jnp.where(m, pos, 0)
            ploc = d16 & (HALF - 1)
            plsc.store_scatter(srcf, [pos], s16, mask=m)
            plsc.store_scatter(rowf, [pos], ploc, mask=m)
            return wp + jnp.max(cs)
        wp = lax.fori_loop(0, EPT // 16, cb, jnp.int32(0))

        iot = lax.iota(jnp.int32, 16)
        cvec = counts_v[...]
        if r == 0:
            cvec = zi
        counts_v[...] = jnp.where(iot == r, wp, cvec)

        def rl(j, _):
            for k in range(8):
                rowl2[j, pl.ds(k * 16, 16)] = rowf[pl.ds(j * CHUNK + k * 16, 16)]
            return 0
        lax.fori_loop(0, NCHUNKS, rl, 0)

        pltpu.sync_copy(
            srcf,
            srcl_hbm.at[pl.ds(((cid * NS + sid) * NREL + r) * CAP, CAP)])
        pltpu.sync_copy(rowl2, rowl_hbm.at[cid, sid, r])

    pltpu.sync_copy(counts_v, counts_hbm.at[pl.ds((cid * NS + sid) * 16, 16)])

    pltpu.sync_copy(hist_v, hshared.at[pl.ds(sid * HKEYS, HKEYS)])
    plsc.subcore_barrier()
    pltpu.sync_copy(hshared.at[pl.ds(sid * HHALF, HHALF)], acc_v)
    for j in range(1, NS):
        pltpu.sync_copy(hshared.at[pl.ds(j * HKEYS + sid * HHALF, HHALF)],
                        tmp_v)

        def ab(i, _):
            sl = pl.ds(i * 16, 16)
            acc_v[sl] = acc_v[sl] + tmp_v[sl]
            return 0
        lax.fori_loop(0, HHALF // 16, ab, 0)

    onef = jnp.ones((16,), jnp.float32)

    def rb(i, _):
        sl = pl.ds(i * 16, 16)
        acc_v[sl] = onef / jnp.maximum(acc_v[sl], onef)
        return 0
    lax.fori_loop(0, HHALF // 16, rb, 0)

    pltpu.sync_copy(acc_v, rcp_hbm.at[pl.ds(cid * HKEYS + sid * HHALF, HHALF)])


@jax.jit
def _preprocess(src, dst, et):
    fn = pl.kernel(
        _preprocess_body,
        out_type=(
            jax.ShapeDtypeStruct((NC * NS * NREL * CAP,), jnp.int32),
            jax.ShapeDtypeStruct((NC, NS, NREL, NCHUNKS, CHUNK), jnp.int32),
            jax.ShapeDtypeStruct((NC * NS * 16,), jnp.int32),
            jax.ShapeDtypeStruct((NC * HKEYS,), jnp.float32),
        ),
        mesh=_mesh(),
        compiler_params=pltpu.CompilerParams(needs_layout_passes=False),
        scratch_types=[
            pltpu.VMEM((EPT,), jnp.int32),
            pltpu.VMEM((EPT,), jnp.int32),
            pltpu.VMEM((EPT,), jnp.int32),
            pltpu.VMEM((HKEYS,), jnp.float32),
            pltpu.VMEM((CAP,), jnp.int32),
            pltpu.VMEM((CAP,), jnp.int32),
            pltpu.VMEM((NCHUNKS, CHUNK), jnp.int32),
            pltpu.VMEM((16,), jnp.int32),
            pltpu.VMEM((HHALF,), jnp.float32),
            pltpu.VMEM((HHALF,), jnp.float32),
            pltpu.VMEM_SHARED((NS * HKEYS,), jnp.float32),
        ],
    )
    return fn(src, dst, et)


def _scatter_body(x_hbm, srcl_hbm, rowl_hbm, counts_hbm, t_hbm,
                  srcl_v, rowl_v, counts_v, rowbuf, zbuf, tbl, gsem, ssem):
    cid = lax.axis_index("c")
    sid = lax.axis_index("s")

    pltpu.sync_copy(counts_hbm.at[pl.ds((cid * NS + sid) * 16, 16)], counts_v)
    _zero_rows(zbuf, 64)
    cv = counts_v[...]

    for r in range(NREL):
        pltpu.sync_copy(
            srcl_hbm.at[pl.ds(((cid * NS + sid) * NREL + r) * CAP, CAP)],
            srcl_v)
        pltpu.sync_copy(rowl_hbm.at[cid, sid, r], rowl_v)
        for b in range(RPT // 64):
            pltpu.sync_copy(zbuf, tbl.at[pl.ds(sid * RPT + b * 64, 64)])
        plsc.subcore_barrier()

        n = cv[r]
        nch = (n + CHUNK - 1) // CHUNK

        def sb(j, _):
            pltpu.async_copy(x_hbm.at[srcl_v.at[pl.ds(j * CHUNK, CHUNK)]],
                             rowbuf, gsem).wait()
            pltpu.async_copy(rowbuf, tbl.at[rowl_v.at[j]], ssem,
                             add=True).wait()
            return 0
        lax.fori_loop(0, nch, sb, 0)
        plsc.subcore_barrier()

        pltpu.sync_copy(tbl.at[pl.ds(sid * RPT, RPT)],
                        t_hbm.at[r, pl.ds(cid * HALF + sid * RPT, RPT)])


@jax.jit
def _sc_scatter(x, srcl, rowl, counts):
    fn = pl.kernel(
        _scatter_body,
        out_type=jax.ShapeDtypeStruct((NREL, N, HID), jnp.float32),
        mesh=_mesh(),
        compiler_params=pltpu.CompilerParams(needs_layout_passes=False),
        scratch_types=[
            pltpu.VMEM((CAP,), jnp.int32),
            pltpu.VMEM((NCHUNKS, CHUNK), jnp.int32),
            pltpu.VMEM((16,), jnp.int32),
            pltpu.VMEM((CHUNK, HID), jnp.float32),
            pltpu.VMEM((64, HID), jnp.float32),
            pltpu.VMEM_SHARED((HALF + 8, HID), jnp.float32),
            pltpu.SemaphoreType.DMA,
            pltpu.SemaphoreType.DMA,
        ],
    )
    return fn(x, srcl, rowl, counts)


def _pool_body(x_hbm, batch_hbm, pool_hbm,
               xbuf, keys1, keys2, zbuf, ptab, ssem):
    cid = lax.axis_index("c")
    sid = lax.axis_index("s")
    rows0 = cid * HALF + sid * (HALF // NS)

    pltpu.sync_copy(x_hbm.at[pl.ds(rows0, 256)], xbuf)
    pltpu.sync_copy(batch_hbm.at[pl.ds(rows0, 256)], keys1)
    for j in range(2):
        for k in range(8):
            keys2[j, pl.ds(k * 16, 16)] = keys1[pl.ds(j * 128 + k * 16, 16)]

    _zero_rows(zbuf, 64)
    pltpu.sync_copy(zbuf, ptab.at[pl.ds(sid * PPT, PPT)])
    plsc.subcore_barrier()

    for j in range(2):
        pltpu.async_copy(xbuf.at[pl.ds(j * 128, 128)],
                         ptab.at[keys2.at[j]], ssem, add=True).wait()
    plsc.subcore_barrier()

    pltpu.sync_copy(ptab.at[pl.ds(sid * PPT, PPT)],
                    pool_hbm.at[cid, pl.ds(sid * PPT, PPT)])


@jax.jit
def _sc_pool(x, batch):
    fn = pl.kernel(
        _pool_body,
        out_type=jax.ShapeDtypeStruct((NC, BS, HID), jnp.float32),
        mesh=_mesh(),
        compiler_params=pltpu.CompilerParams(needs_layout_passes=False),
        scratch_types=[
            pltpu.VMEM((256, HID), jnp.float32),
            pltpu.VMEM((256,), jnp.int32),
            pltpu.VMEM((2, 128), jnp.int32),
            pltpu.VMEM((64, HID), jnp.float32),
            pltpu.VMEM_SHARED((BS, HID), jnp.float32),
            pltpu.SemaphoreType.DMA,
        ],
    )
    return fn(x, batch)


def _weights_kernel(bases_ref, comp_ref, wc_ref):
    for r in range(NREL):
        acc = comp_ref[0, r, 0] * bases_ref[0, 0]
        for b in range(1, NBASES):
            acc = acc + comp_ref[0, r, b] * bases_ref[0, b]
        wc_ref[0, r] = acc


@jax.jit
def _weights(bases_all, comp_all):
    return pl.pallas_call(
        _weights_kernel,
        grid=(3,),
        in_specs=[
            pl.BlockSpec((1, NBASES, HID, HID), lambda l: (l, 0, 0, 0)),
            pl.BlockSpec((1, NREL, NBASES), lambda l: (l, 0, 0),
                         memory_space=pltpu.SMEM),
        ],
        out_specs=pl.BlockSpec((1, NREL, HID, HID), lambda l: (l, 0, 0, 0)),
        out_shape=jax.ShapeDtypeStruct((3, NREL, HID, HID), jnp.float32),
    )(bases_all, comp_all)


def _prestage_kernel(ne_ref, rm_ref, rp_ref, vp_ref, vn_ref, pp_ref, np_ref,
                     ve_ref, wrel_ref, brel_ref, wp_ref, bp_ref, wn_ref,
                     bn_ref, wo_ref, bo_ref, o_ref):
    rel_emb = jnp.dot(rm_ref[...], wrel_ref[...],
                      preferred_element_type=jnp.float32) + brel_ref[...]
    ne = ne_ref[...]
    outs = []
    for i in range(NODE_NUM):
        row = rel_emb[0] * rp_ref[0, i] + rel_emb[1] * rp_ref[1, i]
        c0 = vp_ref[0, i] + vn_ref[0, i]
        c1 = vp_ref[1, i] + vn_ref[1, i]
        emb_i = ne[0, i] * c0 + ne[1, i] * c1 + row[None, :]
        wi = (wp_ref[...] * pp_ref[i] + wn_ref[...] * np_ref[i]
              + wo_ref[...] * ve_ref[i])
        bi = (bp_ref[...] * pp_ref[i] + bn_ref[...] * np_ref[i]
              + bo_ref[...] * ve_ref[i])
        outs.append(jnp.dot(emb_i, wi, preferred_element_type=jnp.float32)
                    + bi)
    x = jnp.stack(outs, axis=1)
    o_ref[...] = x.reshape(NODE_NUM * 128, EMB)


@jax.jit
def _prestage(node_embeds, rel_mats, rel_pos, vec_p_pos, vec_n_pos,
              p_pos, n_pos, vec_e_pos, W_rel, b_rel,
              W_pos, b_pos, W_neg, b_neg, W_oth, b_oth):
    full = lambda shape: pl.BlockSpec(shape, lambda b: tuple(0 for _ in shape))
    smem = lambda shape: pl.BlockSpec(shape, lambda b: tuple(0 for _ in shape),
                                      memory_space=pltpu.SMEM)
    return pl.pallas_call(
        _prestage_kernel,
        grid=(BS // 128,),
        in_specs=[
            pl.BlockSpec((A, NODE_NUM, 128, EMB), lambda b: (0, 0, b, 0)),
            full((A, EMB)),
            smem((A, NODE_NUM)), smem((A, NODE_NUM)), smem((A, NODE_NUM)),
            smem((NODE_NUM,)), smem((NODE_NUM,)), smem((NODE_NUM,)),
            full((EMB, EMB)), full((1, EMB)),
            full((EMB, HID)), full((1, HID)),
            full((EMB, HID)), full((1, HID)),
            full((EMB, HID)), full((1, HID)),
        ],
        out_specs=pl.BlockSpec((NODE_NUM * 128, EMB), lambda b: (b, 0)),
        out_shape=jax.ShapeDtypeStruct((N, HID), jnp.float32),
    )(node_embeds, rel_mats, rel_pos, vec_p_pos, vec_n_pos, p_pos, n_pos,
      vec_e_pos, W_rel, b_rel.reshape(1, EMB), W_pos, b_pos.reshape(1, HID),
      W_neg, b_neg.reshape(1, HID), W_oth, b_oth.reshape(1, HID))


def _layer_kernel(t_ref, x_ref, rcp_ref, wc_ref, root_ref, bias_ref, o_ref,
                  *, relu):
    acc = jnp.dot(x_ref[...], root_ref[...],
                  preferred_element_type=jnp.float32)
    for r in range(NREL):
        part = jnp.dot(t_ref[r], wc_ref[r], preferred_element_type=jnp.float32)
        acc = acc + part * rcp_ref[:, r:r + 1]
    acc = acc + bias_ref[...]
    o_ref[...] = jnp.maximum(acc, 0.0) if relu else acc


@functools.partial(jax.jit, static_argnames=("relu",))
def _layer(t, x, rcp, wc, root, bias, relu):
    MT = 512
    full = lambda shape: pl.BlockSpec(shape, lambda m: tuple(0 for _ in shape))
    return pl.pallas_call(
        functools.partial(_layer_kernel, relu=relu),
        grid=(N // MT,),
        in_specs=[
            pl.BlockSpec((NREL, MT, HID), lambda m: (0, m, 0)),
            pl.BlockSpec((MT, HID), lambda m: (m, 0)),
            pl.BlockSpec((MT, 8), lambda m: (m, 0)),
            full((NREL, HID, HID)),
            full((HID, HID)),
            full((1, HID)),
        ],
        out_specs=pl.BlockSpec((MT, HID), lambda m: (m, 0)),
        out_shape=jax.ShapeDtypeStruct((N, HID), jnp.float32),
    )(t, x, rcp, wc, root, bias.reshape(1, HID))


def _final_kernel(pp_ref, tg_ref, wre_ref, bre_ref, o_ref):
    pooled = pp_ref[0] + pp_ref[1]
    tgt = tg_ref[...]
    t2 = lax.dot_general(tgt, wre_ref[...], (((1,), (1,)), ((), ())),
                         preferred_element_type=jnp.float32)
    s = jnp.sum(pooled * t2, axis=1) + jnp.sum(tgt * bre_ref[...], axis=1)
    o_ref[...] = s[None, :]


@jax.jit
def _final(pools, targets, W_re, b_re):
    full = lambda shape: pl.BlockSpec(shape, lambda: tuple(0 for _ in shape))
    return pl.pallas_call(
        _final_kernel,
        in_specs=[
            full((NC, BS, HID)),
            full((BS, EMB)),
            full((HID, EMB)),
            full((1, EMB)),
        ],
        out_specs=full((1, BS)),
        out_shape=jax.ShapeDtypeStruct((1, BS), jnp.float32),
    )(pools, targets, W_re, b_re.reshape(1, EMB))


def _preprocess_jnp(src, dst, et):
    key = et * N + dst
    cnt = jnp.zeros((NREL * N,), jnp.float32).at[key].add(1.0)
    rcp = 1.0 / jnp.maximum(cnt, 1.0)
    rcp8 = jnp.zeros((N, 8), jnp.float32).at[:, :NREL].set(
        rcp.reshape(NREL, N).T)
    return src, dst, et, rcp8

def _sc_scatter_jnp(x, src, dst, et):
    key = et * N + dst
    t = jax.ops.segment_sum(x[src], key, num_segments=NREL * N)
    return t.reshape(NREL, N, HID)

def _sc_pool_jnp(x, batch):
    p = jax.ops.segment_sum(x, batch, num_segments=BS)
    return jnp.stack([p, jnp.zeros_like(p)])


def kernel(node_embeds, rel_mats, rel_pos, vec_p_pos, vec_n_pos, p_pos, n_pos,
           vec_e_pos, targets_embeds,
           W_rel, b_rel, W_pos, b_pos, W_neg, b_neg, W_oth, b_oth, W_re, b_re,
           bases1, comp1, root1, bias1,
           bases2, comp2, root2, bias2,
           bases3, comp3, root3, bias3,
           edge_index, edge_type, batch):
    src = edge_index[0]
    dst = edge_index[1]
    srcl, rowl, counts, rcp = _preprocess_jnp(src, dst, edge_type)
    wc_all = _weights(jnp.stack([bases1, bases2, bases3]),
                      jnp.stack([comp1, comp2, comp3]))
    x = _prestage(node_embeds, rel_mats, rel_pos, vec_p_pos, vec_n_pos,
                  p_pos, n_pos, vec_e_pos, W_rel, b_rel,
                  W_pos, b_pos, W_neg, b_neg, W_oth, b_oth)
    layers = [(root1, bias1, True), (root2, bias2, True),
              (root3, bias3, False)]
    for li, (root, bias, relu) in enumerate(layers):
        t = _sc_scatter_jnp(x, srcl, rowl, counts)
        x = _layer(t, x, rcp, wc_all[li], root, bias, relu=relu)
    pools = _sc_pool_jnp(x, batch)
    score = _final(pools, targets_embeds, W_re, b_re)
    return score.reshape(BS)
